# baseline (speedup 1.0000x reference)
import dataclasses
import functools

import jax
import jax.numpy as jnp
from jax import lax
from jax.experimental import pallas as pl
from jax.experimental.pallas import tpu as pltpu
from jax.experimental.pallas import tpu_sc as plsc

N = 10000
E = 320000
D = 128
G = 16

NC = 2
NS = 16
NW = NC * NS
EPW = E // NW
WWIN = 128
NWIN = 80
EPWP = NWIN * WWIN
PAD = EPWP - EPW
NSINK = 8
CH = 20
NCH = NWIN // CH
NBUF = 2
STRIPE = 632
LAST0 = 15 * STRIPE
LASTN = N - LAST0

_MESH = plsc.VectorSubcoreMesh(core_axis_name="c", subcore_axis_name="s")
_PREC = lax.Precision.HIGHEST

_SC_PARAMS = pltpu.CompilerParams()
if "needs_layout_passes" in pltpu.CompilerParams.__dataclass_fields__:
    _SC_PARAMS = dataclasses.replace(_SC_PARAMS, needs_layout_passes=False)


@functools.partial(
    pl.kernel,
    out_type=jax.ShapeDtypeStruct((NW, N), jnp.float32),
    mesh=_MESH,
    scratch_types=[
        pltpu.VMEM((EPW,), jnp.int32),
        pltpu.VMEM((N,), jnp.float32),
    ],
    compiler_params=_SC_PARAMS,
)
def _deg_kernel(dst_hbm, degp_hbm, idx_v, hist_v):
    c = lax.axis_index("c")
    s = lax.axis_index("s")
    wid = c * NS + s
    pltpu.sync_copy(dst_hbm.at[wid], idx_v)
    zeros16 = jnp.zeros((16,), jnp.float32)
    ones16 = jnp.ones((16,), jnp.float32)

    @pl.loop(0, N // 16)
    def _(i):
        hist_v[pl.ds(i * 16, 16)] = zeros16

    @pl.loop(0, EPW // 16)
    def _(k):
        idx = idx_v[pl.ds(k * 16, 16)]
        plsc.addupdate_scatter(hist_v, [idx], ones16)

    pltpu.sync_copy(hist_v, degp_hbm.at[wid])


def _make_agg_kernel(nbuf, wwin, nwin, ch):
    assert nwin % ch == 0 and ch % nbuf == 0
    nch = nwin // ch
    scratch = [
        pltpu.VMEM((ch, wwin), jnp.int32),
        pltpu.VMEM((ch, wwin), jnp.int32),
    ]
    scratch += [pltpu.VMEM((wwin, D), jnp.float32)] * nbuf
    scratch += [pltpu.VMEM_SHARED((N + NSINK, D), jnp.float32)]
    scratch += [pltpu.SemaphoreType.DMA] * (2 * nbuf)

    @functools.partial(
        pl.kernel,
        out_type=jax.ShapeDtypeStruct((NC, N, D), jnp.float32),
        mesh=_MESH,
        scratch_types=scratch,
    )
    def _agg(hs_hbm, src_hbm, dst_hbm, aggp_hbm, *scr):
        sidx_v, didx_v = scr[0], scr[1]
        rows = scr[2:2 + nbuf]
        agg_sh = scr[2 + nbuf]
        sg = scr[3 + nbuf:3 + 2 * nbuf]
        ss = scr[3 + 2 * nbuf:3 + 3 * nbuf]

        c = lax.axis_index("c")
        s = lax.axis_index("s")
        wid = c * NS + s
        row0 = pl.multiple_of(s * STRIPE, 8)

        @pl.when(s < NS - 1)
        def _():
            pltpu.sync_copy(hs_hbm.at[pl.ds(row0, STRIPE)],
                            agg_sh.at[pl.ds(row0, STRIPE)])

        @pl.when(s == NS - 1)
        def _():
            pltpu.sync_copy(hs_hbm.at[pl.ds(LAST0, LASTN)],
                            agg_sh.at[pl.ds(LAST0, LASTN)])

        plsc.subcore_barrier()

        def _start_g(w, b):
            pltpu.async_copy(hs_hbm.at[sidx_v.at[w]], rows[b], sg[b])

        def _wait_g(b):
            pltpu.make_async_copy(hs_hbm.at[pl.ds(0, wwin)], rows[b],
                                  sg[b]).wait()

        def _start_s(w, b):
            pltpu.async_copy(rows[b], agg_sh.at[didx_v.at[w]], ss[b],
                             add=True)

        def _wait_s(b):
            pltpu.make_async_copy(rows[b], agg_sh.at[pl.ds(0, wwin)],
                                  ss[b]).wait()

        @pl.loop(0, nch)
        def _(k):
            pltpu.sync_copy(src_hbm.at[wid, k], sidx_v)
            pltpu.sync_copy(dst_hbm.at[wid, k], didx_v)
            for b in range(nbuf - 1):
                _start_g(b, b)
            _wait_g(0)
            _start_s(0, 0)
            _start_g(nbuf - 1, nbuf - 1)

            @pl.loop(0, (ch - nbuf) // nbuf)
            def _(g):
                for j in range(nbuf):
                    w = nbuf * g + 1 + j
                    b = (1 + j) % nbuf
                    _wait_g(b)
                    _start_s(w, b)
                    _wait_s((b + nbuf - 1) % nbuf)
                    _start_g(w + nbuf - 1, (b + nbuf - 1) % nbuf)

            for j in range(nbuf - 1):
                w = ch - (nbuf - 1) + j
                _wait_g(w % nbuf)
                _start_s(w, w % nbuf)
            for b in range(nbuf):
                _wait_s(b)

        plsc.subcore_barrier()

        @pl.when(s < NS - 1)
        def _():
            pltpu.sync_copy(agg_sh.at[pl.ds(row0, STRIPE)],
                            aggp_hbm.at[c, pl.ds(row0, STRIPE)])

        @pl.when(s == NS - 1)
        def _():
            pltpu.sync_copy(agg_sh.at[pl.ds(LAST0, LASTN)],
                            aggp_hbm.at[c, pl.ds(LAST0, LASTN)])

    return _agg


_agg_kernel = _make_agg_kernel(NBUF, WWIN, NWIN, CH)


def _matmul_body(x_ref, w_ref, o_ref):
    o_ref[...] = jnp.dot(x_ref[...], w_ref[...],
                         preferred_element_type=jnp.float32, precision=_PREC)


def _dinv_scale_body(degp_ref, h1_ref, dinv_ref, hs1_ref):
    deg = jnp.sum(degp_ref[...], axis=0) + 1.0
    dinv = lax.rsqrt(jnp.maximum(deg, 1e-12))
    dinv_ref[...] = dinv[:, None]
    hs1_ref[...] = h1_ref[...] * dinv[:, None]


RB = 2000


def _combine_body(aggp_ref, hs_ref, dinv_ref, b_ref, w_ref, hs_next_ref):
    agg = aggp_ref[0] + aggp_ref[1] - hs_ref[...]
    h = jnp.maximum(agg * dinv_ref[...] + b_ref[...], 0.0)
    hn = jnp.dot(h, w_ref[...], preferred_element_type=jnp.float32,
                 precision=_PREC)
    hs_next_ref[...] = hn * dinv_ref[...]


def _combine_call(aggp, hs, dinv, b, w):
    return pl.pallas_call(
        _combine_body,
        grid=(N // RB,),
        in_specs=[
            pl.BlockSpec((NC, RB, D), lambda i: (0, i, 0)),
            pl.BlockSpec((RB, D), lambda i: (i, 0)),
            pl.BlockSpec((RB, 1), lambda i: (i, 0)),
            pl.BlockSpec((1, D), lambda i: (0, 0)),
            pl.BlockSpec((D, D), lambda i: (0, 0)),
        ],
        out_specs=pl.BlockSpec((RB, D), lambda i: (i, 0)),
        out_shape=jax.ShapeDtypeStruct((N, D), jnp.float32),
    )(aggp, hs, dinv, b, w)


def _final_body(aggp_ref, hs_ref, dinv_ref, b_ref, batch_ref, wm_ref, bm_ref,
                out_ref, sums_ref, cnt_ref):
    i = pl.program_id(0)

    @pl.when(i == 0)
    def _():
        sums_ref[...] = jnp.zeros_like(sums_ref)
        cnt_ref[...] = jnp.zeros_like(cnt_ref)

    agg = aggp_ref[0] + aggp_ref[1] - hs_ref[...]
    h3 = agg * dinv_ref[...] + b_ref[...]
    seg = batch_ref[...][:, 0]
    onehot = (seg[None, :] == lax.broadcasted_iota(jnp.int32, (G, RB), 0))
    onehot = onehot.astype(jnp.float32)
    sums_ref[...] += jnp.dot(onehot, h3, preferred_element_type=jnp.float32,
                             precision=_PREC)
    cnt_ref[...] += jnp.sum(onehot, axis=1, keepdims=True)

    @pl.when(i == pl.num_programs(0) - 1)
    def _():
        pooled = sums_ref[...] / jnp.maximum(cnt_ref[...], 1.0)
        out_ref[...] = jnp.dot(pooled, wm_ref[...],
                               preferred_element_type=jnp.float32,
                               precision=_PREC) + bm_ref[...]


def _final_call(aggp, hs, dinv, b, batch32, wm, bm):
    return pl.pallas_call(
        _final_body,
        grid=(N // RB,),
        in_specs=[
            pl.BlockSpec((NC, RB, D), lambda i: (0, i, 0)),
            pl.BlockSpec((RB, D), lambda i: (i, 0)),
            pl.BlockSpec((RB, 1), lambda i: (i, 0)),
            pl.BlockSpec((1, D), lambda i: (0, 0)),
            pl.BlockSpec((RB, 1), lambda i: (i, 0)),
            pl.BlockSpec((D, D), lambda i: (0, 0)),
            pl.BlockSpec((1, D), lambda i: (0, 0)),
        ],
        out_specs=pl.BlockSpec((G, D), lambda i: (0, 0)),
        out_shape=jax.ShapeDtypeStruct((G, D), jnp.float32),
        scratch_shapes=[
            pltpu.VMEM((G, D), jnp.float32),
            pltpu.VMEM((G, 1), jnp.float32),
        ],
    )(aggp, hs, dinv, b, batch32, wm, bm)


def _tc_call(body, out_shapes, *args):
    return pl.pallas_call(body, out_shape=out_shapes)(*args)


def kernel(x, adj, batch, W1, b1, W2, b2, W3, b3, Wm, bm):
    src = adj[0].astype(jnp.int32)
    dst = adj[1].astype(jnp.int32)
    padv = (jnp.arange(PAD, dtype=jnp.int32) % NSINK)[None, :]
    pad_src = jnp.tile(padv, (NW, 1))
    pad_dst = pad_src + N
    srcw = jnp.concatenate([src.reshape(NW, EPW), pad_src],
                           axis=1).reshape(NW, NCH, CH, WWIN)
    dstw = jnp.concatenate([dst.reshape(NW, EPW), pad_dst],
                           axis=1).reshape(NW, NCH, CH, WWIN)
    dstf = dst.reshape(NW, EPW)
    batch32 = batch.astype(jnp.int32).reshape(N, 1)

    f32 = jnp.float32
    degp = _deg_kernel(dstf)
    h1 = _tc_call(_matmul_body, jax.ShapeDtypeStruct((N, D), f32), x, W1)
    dinv, hs1 = _tc_call(
        _dinv_scale_body,
        [jax.ShapeDtypeStruct((N, 1), f32), jax.ShapeDtypeStruct((N, D), f32)],
        degp, h1)

    aggp1 = _agg_kernel(hs1, srcw, dstw)
    hs2 = _combine_call(aggp1, hs1, dinv, b1.reshape(1, D), W2)

    aggp2 = _agg_kernel(hs2, srcw, dstw)
    hs3 = _combine_call(aggp2, hs2, dinv, b2.reshape(1, D), W3)

    aggp3 = _agg_kernel(hs3, srcw, dstw)
    out = _final_call(aggp3, hs3, dinv, b3.reshape(1, D), batch32,
                      Wm, bm.reshape(1, D))
    return out


# speedup vs baseline: 1.3397x; 1.3397x over previous
import dataclasses
import functools

import jax
import jax.numpy as jnp
from jax import lax
from jax.experimental import pallas as pl
from jax.experimental.pallas import tpu as pltpu
from jax.experimental.pallas import tpu_sc as plsc

N = 10000
E = 320000
D = 128
G = 16

NC = 2
NS = 16
NW = NC * NS
EPW = E // NW
WWIN = 80
NWIN = 126
EPWP = NWIN * WWIN
PAD = EPWP - EPW
NSINK = 8
CH = 42
NCH = NWIN // CH
NBUF = 3
STRIPE = 632
LAST0 = 15 * STRIPE
LASTN = N - LAST0

_MESH = plsc.VectorSubcoreMesh(core_axis_name="c", subcore_axis_name="s")
_PREC = lax.Precision.HIGHEST

_SC_PARAMS = pltpu.CompilerParams()
if "needs_layout_passes" in pltpu.CompilerParams.__dataclass_fields__:
    _SC_PARAMS = dataclasses.replace(_SC_PARAMS, needs_layout_passes=False)


@functools.partial(
    pl.kernel,
    out_type=jax.ShapeDtypeStruct((NW, N), jnp.float32),
    mesh=_MESH,
    scratch_types=[
        pltpu.VMEM((EPW,), jnp.int32),
        pltpu.VMEM((N,), jnp.float32),
    ],
    compiler_params=_SC_PARAMS,
)
def _deg_kernel(dst_hbm, degp_hbm, idx_v, hist_v):
    c = lax.axis_index("c")
    s = lax.axis_index("s")
    wid = c * NS + s
    pltpu.sync_copy(dst_hbm.at[wid], idx_v)
    zeros16 = jnp.zeros((16,), jnp.float32)
    ones16 = jnp.ones((16,), jnp.float32)

    @pl.loop(0, N // 16)
    def _(i):
        hist_v[pl.ds(i * 16, 16)] = zeros16

    @pl.loop(0, EPW // 16)
    def _(k):
        idx = idx_v[pl.ds(k * 16, 16)]
        plsc.addupdate_scatter(hist_v, [idx], ones16)

    pltpu.sync_copy(hist_v, degp_hbm.at[wid])


def _make_agg_kernel(nbuf, wwin, nwin, ch):
    assert nwin % ch == 0 and ch % nbuf == 0
    nch = nwin // ch
    scratch = [
        pltpu.VMEM((ch, wwin), jnp.int32),
        pltpu.VMEM((ch, wwin), jnp.int32),
    ]
    scratch += [pltpu.VMEM((wwin, D), jnp.float32)] * nbuf
    scratch += [pltpu.VMEM_SHARED((N + NSINK, D), jnp.float32)]
    scratch += [pltpu.SemaphoreType.DMA] * (2 * nbuf)

    @functools.partial(
        pl.kernel,
        out_type=jax.ShapeDtypeStruct((NC, N, D), jnp.float32),
        mesh=_MESH,
        scratch_types=scratch,
    )
    def _agg(hs_hbm, src_hbm, dst_hbm, aggp_hbm, *scr):
        sidx_v, didx_v = scr[0], scr[1]
        rows = scr[2:2 + nbuf]
        agg_sh = scr[2 + nbuf]
        sg = scr[3 + nbuf:3 + 2 * nbuf]
        ss = scr[3 + 2 * nbuf:3 + 3 * nbuf]

        c = lax.axis_index("c")
        s = lax.axis_index("s")
        wid = c * NS + s
        row0 = pl.multiple_of(s * STRIPE, 8)

        @pl.when(s < NS - 1)
        def _():
            pltpu.sync_copy(hs_hbm.at[pl.ds(row0, STRIPE)],
                            agg_sh.at[pl.ds(row0, STRIPE)])

        @pl.when(s == NS - 1)
        def _():
            pltpu.sync_copy(hs_hbm.at[pl.ds(LAST0, LASTN)],
                            agg_sh.at[pl.ds(LAST0, LASTN)])

        plsc.subcore_barrier()

        def _start_g(w, b):
            pltpu.async_copy(hs_hbm.at[sidx_v.at[w]], rows[b], sg[b])

        def _wait_g(b):
            pltpu.make_async_copy(hs_hbm.at[pl.ds(0, wwin)], rows[b],
                                  sg[b]).wait()

        def _start_s(w, b):
            pltpu.async_copy(rows[b], agg_sh.at[didx_v.at[w]], ss[b],
                             add=True)

        def _wait_s(b):
            pltpu.make_async_copy(rows[b], agg_sh.at[pl.ds(0, wwin)],
                                  ss[b]).wait()

        @pl.loop(0, nch)
        def _(k):
            pltpu.sync_copy(src_hbm.at[wid, k], sidx_v)
            pltpu.sync_copy(dst_hbm.at[wid, k], didx_v)
            for b in range(nbuf - 1):
                _start_g(b, b)
            _wait_g(0)
            _start_s(0, 0)
            _start_g(nbuf - 1, nbuf - 1)

            @pl.loop(0, (ch - nbuf) // nbuf)
            def _(g):
                for j in range(nbuf):
                    w = nbuf * g + 1 + j
                    b = (1 + j) % nbuf
                    _wait_g(b)
                    _start_s(w, b)
                    _wait_s((b + nbuf - 1) % nbuf)
                    _start_g(w + nbuf - 1, (b + nbuf - 1) % nbuf)

            for j in range(nbuf - 1):
                w = ch - (nbuf - 1) + j
                _wait_g(w % nbuf)
                _start_s(w, w % nbuf)
            for b in range(nbuf):
                _wait_s(b)

        plsc.subcore_barrier()

        @pl.when(s < NS - 1)
        def _():
            pltpu.sync_copy(agg_sh.at[pl.ds(row0, STRIPE)],
                            aggp_hbm.at[c, pl.ds(row0, STRIPE)])

        @pl.when(s == NS - 1)
        def _():
            pltpu.sync_copy(agg_sh.at[pl.ds(LAST0, LASTN)],
                            aggp_hbm.at[c, pl.ds(LAST0, LASTN)])

    return _agg


_agg_kernel = _make_agg_kernel(NBUF, WWIN, NWIN, CH)


def _matmul_body(x_ref, w_ref, o_ref):
    o_ref[...] = jnp.dot(x_ref[...], w_ref[...],
                         preferred_element_type=jnp.float32, precision=_PREC)


def _dinv_scale_body(degp_ref, h1_ref, dinv_ref, hs1_ref):
    deg = jnp.sum(degp_ref[...], axis=0) + 1.0
    dinv = lax.rsqrt(jnp.maximum(deg, 1e-12))
    dinv_ref[...] = dinv[:, None]
    hs1_ref[...] = h1_ref[...] * dinv[:, None]


RB = 2000


def _combine_body(aggp_ref, hs_ref, dinv_ref, b_ref, w_ref, hs_next_ref):
    agg = aggp_ref[0] + aggp_ref[1] - hs_ref[...]
    h = jnp.maximum(agg * dinv_ref[...] + b_ref[...], 0.0)
    hn = jnp.dot(h, w_ref[...], preferred_element_type=jnp.float32,
                 precision=_PREC)
    hs_next_ref[...] = hn * dinv_ref[...]


def _combine_call(aggp, hs, dinv, b, w):
    return pl.pallas_call(
        _combine_body,
        grid=(N // RB,),
        in_specs=[
            pl.BlockSpec((NC, RB, D), lambda i: (0, i, 0)),
            pl.BlockSpec((RB, D), lambda i: (i, 0)),
            pl.BlockSpec((RB, 1), lambda i: (i, 0)),
            pl.BlockSpec((1, D), lambda i: (0, 0)),
            pl.BlockSpec((D, D), lambda i: (0, 0)),
        ],
        out_specs=pl.BlockSpec((RB, D), lambda i: (i, 0)),
        out_shape=jax.ShapeDtypeStruct((N, D), jnp.float32),
    )(aggp, hs, dinv, b, w)


def _final_body(aggp_ref, hs_ref, dinv_ref, b_ref, batch_ref, wm_ref, bm_ref,
                out_ref, sums_ref, cnt_ref):
    i = pl.program_id(0)

    @pl.when(i == 0)
    def _():
        sums_ref[...] = jnp.zeros_like(sums_ref)
        cnt_ref[...] = jnp.zeros_like(cnt_ref)

    agg = aggp_ref[0] + aggp_ref[1] - hs_ref[...]
    h3 = agg * dinv_ref[...] + b_ref[...]
    seg = batch_ref[...][:, 0]
    onehot = (seg[None, :] == lax.broadcasted_iota(jnp.int32, (G, RB), 0))
    onehot = onehot.astype(jnp.float32)
    sums_ref[...] += jnp.dot(onehot, h3, preferred_element_type=jnp.float32,
                             precision=_PREC)
    cnt_ref[...] += jnp.sum(onehot, axis=1, keepdims=True)

    @pl.when(i == pl.num_programs(0) - 1)
    def _():
        pooled = sums_ref[...] / jnp.maximum(cnt_ref[...], 1.0)
        out_ref[...] = jnp.dot(pooled, wm_ref[...],
                               preferred_element_type=jnp.float32,
                               precision=_PREC) + bm_ref[...]


def _final_call(aggp, hs, dinv, b, batch32, wm, bm):
    return pl.pallas_call(
        _final_body,
        grid=(N // RB,),
        in_specs=[
            pl.BlockSpec((NC, RB, D), lambda i: (0, i, 0)),
            pl.BlockSpec((RB, D), lambda i: (i, 0)),
            pl.BlockSpec((RB, 1), lambda i: (i, 0)),
            pl.BlockSpec((1, D), lambda i: (0, 0)),
            pl.BlockSpec((RB, 1), lambda i: (i, 0)),
            pl.BlockSpec((D, D), lambda i: (0, 0)),
            pl.BlockSpec((1, D), lambda i: (0, 0)),
        ],
        out_specs=pl.BlockSpec((G, D), lambda i: (0, 0)),
        out_shape=jax.ShapeDtypeStruct((G, D), jnp.float32),
        scratch_shapes=[
            pltpu.VMEM((G, D), jnp.float32),
            pltpu.VMEM((G, 1), jnp.float32),
        ],
    )(aggp, hs, dinv, b, batch32, wm, bm)


def _tc_call(body, out_shapes, *args):
    return pl.pallas_call(body, out_shape=out_shapes)(*args)


def kernel(x, adj, batch, W1, b1, W2, b2, W3, b3, Wm, bm):
    src = adj[0].astype(jnp.int32)
    dst = adj[1].astype(jnp.int32)
    padv = (jnp.arange(PAD, dtype=jnp.int32) % NSINK)[None, :]
    pad_src = jnp.tile(padv, (NW, 1))
    pad_dst = pad_src + N
    srcw = jnp.concatenate([src.reshape(NW, EPW), pad_src],
                           axis=1).reshape(NW, NCH, CH, WWIN)
    dstw = jnp.concatenate([dst.reshape(NW, EPW), pad_dst],
                           axis=1).reshape(NW, NCH, CH, WWIN)
    dstf = dst.reshape(NW, EPW)
    batch32 = batch.astype(jnp.int32).reshape(N, 1)

    f32 = jnp.float32
    degp = _deg_kernel(dstf)
    h1 = _tc_call(_matmul_body, jax.ShapeDtypeStruct((N, D), f32), x, W1)
    dinv, hs1 = _tc_call(
        _dinv_scale_body,
        [jax.ShapeDtypeStruct((N, 1), f32), jax.ShapeDtypeStruct((N, D), f32)],
        degp, h1)

    aggp1 = _agg_kernel(hs1, srcw, dstw)
    hs2 = _combine_call(aggp1, hs1, dinv, b1.reshape(1, D), W2)

    aggp2 = _agg_kernel(hs2, srcw, dstw)
    hs3 = _combine_call(aggp2, hs2, dinv, b2.reshape(1, D), W3)

    aggp3 = _agg_kernel(hs3, srcw, dstw)
    out = _final_call(aggp3, hs3, dinv, b3.reshape(1, D), batch32,
                      Wm, bm.reshape(1, D))
    return out
